# baseline (speedup 1.0000x reference)
import dataclasses
import functools

import jax
import jax.numpy as jnp
from jax.experimental import pallas as pl
from jax.experimental.pallas import tpu as pltpu
from jax.experimental.pallas import tpu_sc as plsc

M_TILE = 2048
NC, NS = 2, 16
LANES = 16
CHUNK = 64


def _argmin_body(x_ref, w_ref, b_ref, o_ref):
    xb = x_ref[...].astype(jnp.bfloat16)
    wb = w_ref[...].astype(jnp.bfloat16)
    dot = jax.lax.dot_general(
        xb, wb, (((1,), (1,)), ((), ())),
        preferred_element_type=jnp.float32)
    dists = -2.0 * dot + b_ref[0, :][None, :]
    o_ref[0, 0, :] = jnp.argmin(dists, axis=1).astype(jnp.int32)


def _tc_argmin(x, weight, bias):
    m, _ = x.shape
    num_blocks = m // M_TILE
    out = pl.pallas_call(
        _argmin_body,
        grid=(num_blocks,),
        in_specs=[
            pl.BlockSpec((M_TILE, x.shape[1]), lambda i: (i, 0)),
            pl.BlockSpec(weight.shape, lambda i: (0, 0)),
            pl.BlockSpec(bias.shape, lambda i: (0, 0)),
        ],
        out_specs=pl.BlockSpec((1, 1, M_TILE), lambda i: (i, 0, 0)),
        out_shape=jax.ShapeDtypeStruct((num_blocks, 1, M_TILE), jnp.int32),
    )(x, weight, bias)
    return out.reshape(m)


def _sc_gather(weight, idxes):
    n = idxes.shape[0]
    vocab, dim = weight.shape
    nw = NC * NS
    b_per_w = n // nw
    nchunks = b_per_w // CHUNK
    groups = CHUNK // LANES
    w_flat = weight.reshape(vocab * dim)
    mesh = plsc.VectorSubcoreMesh(core_axis_name="c", subcore_axis_name="s")

    cp = pltpu.CompilerParams()
    if "needs_layout_passes" in pltpu.CompilerParams.__dataclass_fields__:
        cp = dataclasses.replace(cp, needs_layout_passes=False)

    @functools.partial(
        pl.kernel, mesh=mesh,
        compiler_params=cp,
        out_type=jax.ShapeDtypeStruct((n * dim,), jnp.float32),
        scratch_types=[
            pltpu.VMEM((vocab * dim,), jnp.float32),
            pltpu.VMEM((b_per_w,), jnp.int32),
            pltpu.VMEM((CHUNK * dim,), jnp.float32),
            pltpu.VMEM((CHUNK * dim,), jnp.float32),
            pltpu.SemaphoreType.DMA,
            pltpu.SemaphoreType.DMA,
        ],
    )
    def kern(w_hbm, i_hbm, o_hbm, w_v, idx_v, buf0, buf1, sem0, sem1):
        wid = jax.lax.axis_index("c") * NS + jax.lax.axis_index("s")
        row0 = wid * b_per_w
        pltpu.sync_copy(i_hbm.at[pl.ds(row0 * 1, b_per_w)], idx_v)
        pltpu.sync_copy(w_hbm, w_v)

        lane = jax.lax.iota(jnp.int32, LANES)
        bufs = (buf0, buf1)
        sems = (sem0, sem1)
        handles = [None, None]
        for chunk in range(nchunks):
            par = chunk % 2
            buf = bufs[par]
            if handles[par] is not None:
                handles[par].wait()

            @pl.loop(0, groups)
            def _(g, buf=buf, chunk=chunk):
                jvec = idx_v[pl.ds(chunk * CHUNK + g * LANES, LANES)]
                jbase = jvec * dim
                obase = (g * LANES + lane) * dim

                @pl.loop(0, dim, unroll=8)
                def _(c, jbase=jbase, obase=obase, buf=buf):
                    vals = plsc.load_gather(w_v, [jbase + c])
                    plsc.store_scatter(buf, [obase + c], vals)

            handles[par] = pltpu.async_copy(
                buf,
                o_hbm.at[pl.ds((row0 + chunk * CHUNK) * dim, CHUNK * dim)],
                sems[par])
        for par in range(2):
            if handles[par] is not None:
                handles[par].wait()

    return kern(w_flat, idxes).reshape(n, dim)


def kernel(input, weight):
    embed_dim = input.shape[-1]
    x = input.reshape(-1, embed_dim)
    bias = (weight * weight).sum(0)[None, :]
    idxes = _tc_argmin(x, weight, bias)
    values = _sc_gather(weight, idxes)
    return values.reshape(input.shape)
